# baseline (speedup 1.0000x reference)
import functools

import jax
import jax.numpy as jnp
from jax import lax
from jax.experimental import pallas as pl
from jax.experimental.pallas import tpu as pltpu
from jax.experimental.pallas import tpu_sc as plsc

N, C, H, W = 1000, 66, 28, 28
NC, NS, L = 2, 16, 16
NWORK = NC * NS
NPAD = 1024
CH = 128
CPP = NPAD // CH
NBUF = 12
NTASKS = H * W * CPP
T_PER_W = NTASKS // NWORK
NITER = -(-T_PER_W // NBUF)

_mesh = plsc.VectorSubcoreMesh(core_axis_name="c", subcore_axis_name="s")


@functools.partial(
    pl.kernel,
    mesh=_mesh,
    out_type=jax.ShapeDtypeStruct((3, H, W, N), jnp.float32),
    scratch_types=(
        [pltpu.VMEM((NPAD,), jnp.int32)]
        + [pltpu.VMEM((C, CH), jnp.float32) for _ in range(NBUF)]
        + [pltpu.VMEM((3, CH), jnp.float32) for _ in range(NBUF)]
        + [pltpu.SemaphoreType.DMA for _ in range(2 * NBUF)]
    ),
    compiler_params=pltpu.CompilerParams(
        use_tc_tiling_on_sc=True,
        needs_layout_passes=False,
        disable_bounds_checks=True,
    ),
)
def _select_rows(b_hbm, labels_hbm, out_hbm, lbl_v, *bufs):
    slabs = bufs[:NBUF]
    obufs = bufs[NBUF : 2 * NBUF]
    lsems = bufs[2 * NBUF : 3 * NBUF]
    osems = bufs[3 * NBUF :]

    wid = lax.axis_index("s") * NC + lax.axis_index("c")
    pltpu.sync_copy(labels_hbm, lbl_v.at[pl.ds(0, N)])
    lanes = lax.iota(jnp.int32, L)
    t0 = wid * T_PER_W

    def coords(t):
        p = t // CPP
        return p // W, p % W, (t % CPP) * CH

    def issue_load(t, b):
        h, w, o = coords(t)
        pltpu.async_copy(b_hbm.at[h, w, :, pl.ds(o, CH)], slabs[b], lsems[b])

    def drain_load(b):
        pltpu.make_async_copy(
            b_hbm.at[0, 0, :, pl.ds(0, CH)], slabs[b], lsems[b]
        ).wait()

    def gather(t, b):
        _, _, o = coords(t)
        for g in range(CH // L):
            base = 3 * jnp.clip(lbl_v[pl.ds(o + g * L, L)], 0, C // 3 - 1)
            col = g * L + lanes
            for c in range(3):
                obufs[b][c, pl.ds(g * L, L)] = plsc.load_gather(
                    slabs[b], [base + c, col]
                )

    def issue_out(t, b):
        h, w, o = coords(t)
        pltpu.async_copy(obufs[b], out_hbm.at[:, h, w, pl.ds(o, CH)], osems[b])

    def drain_out(b):
        pltpu.make_async_copy(
            obufs[b], out_hbm.at[:, 0, 0, pl.ds(0, CH)], osems[b]
        ).wait()

    for b in range(NBUF):
        issue_load(t0 + b, b)

    def step(i, carry):
        for b in range(NBUF):
            k = i * NBUF + b

            @pl.when(k < T_PER_W)
            def _(i=i, b=b, k=k):
                t = t0 + k

                @pl.when(i > 0)
                def _():
                    drain_out(b)

                drain_load(b)
                gather(t, b)

                @pl.when(k + NBUF < T_PER_W)
                def _():
                    issue_load(t + NBUF, b)

                issue_out(t, b)

        return carry

    lax.fori_loop(0, NITER, step, None)
    for b in range(NBUF):
        drain_out(b)


def kernel(vert_pred, labels):
    b = jnp.transpose(vert_pred, (2, 3, 1, 0))
    out = _select_rows(b, labels.astype(jnp.int32))
    return jnp.transpose(out, (3, 0, 1, 2))


# speedup vs baseline: 1.0309x; 1.0309x over previous
import functools

import jax
import jax.numpy as jnp
from jax import lax
from jax.experimental import pallas as pl
from jax.experimental.pallas import tpu as pltpu
from jax.experimental.pallas import tpu_sc as plsc

N, C, H, W = 1000, 66, 28, 28
NC, NS, L = 2, 16, 16
NWORK = NC * NS
NPAD = 1024
CH = 128
CPP = NPAD // CH
NBUF = 10
NTASKS = H * W * CPP
T_PER_W = NTASKS // NWORK
NITER = -(-T_PER_W // NBUF)

_mesh = plsc.VectorSubcoreMesh(core_axis_name="c", subcore_axis_name="s")


@functools.partial(
    pl.kernel,
    mesh=_mesh,
    out_type=jax.ShapeDtypeStruct((3, H, W, N), jnp.float32),
    scratch_types=(
        [pltpu.VMEM((NPAD,), jnp.int32)]
        + [pltpu.VMEM((C, CH), jnp.float32) for _ in range(NBUF)]
        + [pltpu.VMEM((3, CH), jnp.float32) for _ in range(NBUF)]
        + [pltpu.SemaphoreType.DMA for _ in range(2 * NBUF)]
    ),
    compiler_params=pltpu.CompilerParams(
        use_tc_tiling_on_sc=True,
        needs_layout_passes=False,
        disable_bounds_checks=True,
    ),
)
def _select_rows(b_hbm, labels_hbm, out_hbm, lbl_v, *bufs):
    slabs = bufs[:NBUF]
    obufs = bufs[NBUF : 2 * NBUF]
    lsems = bufs[2 * NBUF : 3 * NBUF]
    osems = bufs[3 * NBUF :]

    wid = lax.axis_index("s") * NC + lax.axis_index("c")
    pltpu.sync_copy(labels_hbm, lbl_v.at[pl.ds(0, N)])
    lanes = lax.iota(jnp.int32, L)
    t0 = wid * T_PER_W

    def coords(t):
        p = t // CPP
        return p // W, p % W, (t % CPP) * CH

    def issue_load(t, b):
        h, w, o = coords(t)
        pltpu.async_copy(b_hbm.at[h, w, :, pl.ds(o, CH)], slabs[b], lsems[b])

    def drain_load(b):
        pltpu.make_async_copy(
            b_hbm.at[0, 0, :, pl.ds(0, CH)], slabs[b], lsems[b]
        ).wait()

    def gather(t, b):
        _, _, o = coords(t)
        for g in range(CH // L):
            base = 3 * jnp.clip(lbl_v[pl.ds(o + g * L, L)], 0, C // 3 - 1)
            col = g * L + lanes
            for c in range(3):
                obufs[b][c, pl.ds(g * L, L)] = plsc.load_gather(
                    slabs[b], [base + c, col]
                )

    def issue_out(t, b):
        h, w, o = coords(t)
        pltpu.async_copy(obufs[b], out_hbm.at[:, h, w, pl.ds(o, CH)], osems[b])

    def drain_out(b):
        pltpu.make_async_copy(
            obufs[b], out_hbm.at[:, 0, 0, pl.ds(0, CH)], osems[b]
        ).wait()

    for b in range(NBUF):
        issue_load(t0 + b, b)

    def step(i, carry):
        for b in range(NBUF):
            k = i * NBUF + b

            @pl.when(k < T_PER_W)
            def _(i=i, b=b, k=k):
                t = t0 + k

                @pl.when(i > 0)
                def _():
                    drain_out(b)

                drain_load(b)
                gather(t, b)

                @pl.when(k + NBUF < T_PER_W)
                def _():
                    issue_load(t + NBUF, b)

                issue_out(t, b)

        return carry

    lax.fori_loop(0, NITER, step, None)
    for b in range(NBUF):
        drain_out(b)


def kernel(vert_pred, labels):
    b = jnp.transpose(vert_pred, (2, 3, 1, 0))
    out = _select_rows(b, labels.astype(jnp.int32))
    return jnp.transpose(out, (3, 0, 1, 2))
